# baseline (speedup 1.0000x reference)
import functools

import jax
import jax.numpy as jnp
from jax import lax
from jax.experimental import pallas as pl
from jax.experimental.pallas import tpu as pltpu
from jax.experimental.pallas import tpu_sc as plsc

N = 10000
D = 128
NUM_REL_ROWS = 200
REL_ROWS = NUM_REL_ROWS + 8
ZROW = NUM_REL_ROWS

NC, NS = 2, 16
CH = 128
T_CHUNKS = 160
SB = 80
TOT_CHUNKS = NC * NS * T_CHUNKS
NREAL_CHUNKS = 5000
ACC_ROWS = 10112
ROWS_PER_TILE = ACC_ROWS // NS
NEC = 2500


def _sc_scatter(x, negrel, ei3, et2, zblk):
    mesh = plsc.VectorSubcoreMesh(
        core_axis_name="c", subcore_axis_name="s", num_cores=NC, num_subcores=NS)

    @functools.partial(
        pl.kernel,
        out_type=jax.ShapeDtypeStruct((NC, ACC_ROWS, D), jnp.float32),
        mesh=mesh,
        scratch_types=[
            pltpu.VMEM((SB // 2, CH), jnp.int32),
            pltpu.VMEM((SB // 2, CH), jnp.int32),
            pltpu.VMEM((SB // 2, CH), jnp.int32),
            pltpu.VMEM((CH, D), jnp.float32),
            pltpu.VMEM((CH, D), jnp.float32),
            pltpu.VMEM_SHARED((ACC_ROWS, D), jnp.float32),
            pltpu.VMEM_SHARED((REL_ROWS, D), jnp.float32),
            pltpu.SemaphoreType.DMA,
            pltpu.SemaphoreType.DMA,
        ],
    )
    def k(x_hbm, negrel_hbm, ei_hbm, et_hbm, z_hbm, out_hbm,
          srcv, etv, dstv, rows0, rows1, acc, nrel, sem0, sem1):
        c = lax.axis_index("c")
        s = lax.axis_index("s")
        wid = c * NS + s
        base = wid * T_CHUNKS

        pltpu.sync_copy(z_hbm, acc.at[pl.ds(s * ROWS_PER_TILE, ROWS_PER_TILE)])

        @pl.when(s == 0)
        def _():
            pltpu.sync_copy(negrel_hbm, nrel)

        plsc.subcore_barrier()

        rows = (rows0, rows1)
        sems = (sem0, sem1)
        gtab = (srcv, etv)

        def issue_gather(stage_base, q, b):
            @pl.when(stage_base + 2 * q + b < NREAL_CHUNKS)
            def _():
                tbl = x_hbm if b == 0 else nrel
                pltpu.async_copy(tbl.at[gtab[b].at[q]], rows[b], sems[b])

        @pl.loop(0, T_CHUNKS // SB)
        def _stage(j):
            stage_base = base + j * SB

            @pl.when(stage_base < NREAL_CHUNKS)
            def _():
                pair_base = wid * (T_CHUNKS // 2) + j * (SB // 2)
                pltpu.sync_copy(ei_hbm.at[0, pl.ds(pair_base, SB // 2)], srcv)
                pltpu.sync_copy(et_hbm.at[pl.ds(pair_base, SB // 2)], etv)
                pltpu.sync_copy(ei_hbm.at[1, pl.ds(pair_base, SB // 2)], dstv)
                issue_gather(stage_base, 0, 0)
                issue_gather(stage_base, 0, 1)

                @pl.loop(0, SB // 2, unroll=4)
                def _pair(q):
                    for b in range(2):
                        @pl.when(stage_base + 2 * q + b < NREAL_CHUNKS)
                        def _():
                            pltpu.make_async_copy(
                                x_hbm.at[pl.ds(0, CH)], rows[b],
                                sems[b]).wait()
                            pltpu.sync_copy(rows[b], acc.at[dstv.at[q]],
                                            add=True)

                            @pl.when(q + 1 < SB // 2)
                            def _():
                                issue_gather(stage_base, q + 1, b)

        plsc.subcore_barrier()
        pltpu.sync_copy(
            acc.at[pl.ds(s * ROWS_PER_TILE, ROWS_PER_TILE)],
            out_hbm.at[c, pl.ds(s * ROWS_PER_TILE, ROWS_PER_TILE)])

    return k(x, negrel, ei3, et2, zblk)


def _tc_matmuls(x, acc, weight, bias2d, rel_pad, rel_weight):
    BM = 1000

    def body(x_ref, a0_ref, a1_ref, w_ref, b_ref, r_ref, rw_ref,
             o_ref, ro_ref):
        s = x_ref[...] + a0_ref[0] + a1_ref[0]
        o_ref[...] = jnp.dot(s, w_ref[...],
                             preferred_element_type=jnp.float32) + b_ref[...]
        ro_ref[...] = jnp.dot(r_ref[...], rw_ref[...],
                              preferred_element_type=jnp.float32)

    return pl.pallas_call(
        body,
        grid=(N // BM,),
        in_specs=[
            pl.BlockSpec((BM, D), lambda i: (i, 0)),
            pl.BlockSpec((1, BM, D), lambda i: (0, i, 0)),
            pl.BlockSpec((1, BM, D), lambda i: (1, i, 0)),
            pl.BlockSpec((D, D), lambda i: (0, 0)),
            pl.BlockSpec((1, D), lambda i: (0, 0)),
            pl.BlockSpec((208, D), lambda i: (0, 0)),
            pl.BlockSpec((D, D), lambda i: (0, 0)),
        ],
        out_specs=[
            pl.BlockSpec((BM, D), lambda i: (i, 0)),
            pl.BlockSpec((208, D), lambda i: (0, 0)),
        ],
        out_shape=[
            jax.ShapeDtypeStruct((N, D), jnp.float32),
            jax.ShapeDtypeStruct((208, D), jnp.float32),
        ],
    )(x, acc, acc, weight, bias2d, rel_pad, rel_weight)


def kernel(x, edge_index, edge_type, rel_embed, weight, rel_weight, bias):
    negrel = jnp.concatenate(
        [-rel_embed, jnp.zeros((REL_ROWS - NUM_REL_ROWS, D), jnp.float32)])

    ei3 = edge_index.reshape(2, NEC, CH)
    et2 = edge_type.reshape(NEC, CH)
    zblk = jnp.zeros((ROWS_PER_TILE, D), jnp.float32)
    acc = _sc_scatter(x, negrel, ei3, et2, zblk)

    rel_pad = jnp.concatenate([rel_embed, jnp.zeros((8, D), jnp.float32)])
    out, rel_out = _tc_matmuls(x, acc, weight, bias.reshape(1, D),
                               rel_pad, rel_weight)
    return (out, rel_out[:NUM_REL_ROWS + 1])


# speedup vs baseline: 1.0375x; 1.0375x over previous
import functools

import jax
import jax.numpy as jnp
from jax import lax
from jax.experimental import pallas as pl
from jax.experimental.pallas import tpu as pltpu
from jax.experimental.pallas import tpu_sc as plsc

N = 10000
D = 128
NUM_REL_ROWS = 200
REL_ROWS = NUM_REL_ROWS + 8
ZROW = NUM_REL_ROWS

NC, NS = 2, 16
CH = 128
T_CHUNKS = 160
SB = 80
TOT_CHUNKS = NC * NS * T_CHUNKS
NREAL_CHUNKS = 5000
ACC_ROWS = 10112
ROWS_PER_TILE = ACC_ROWS // NS
NEC = 2500


def _sc_scatter(x, negrel, ei3, et2, zblk):
    mesh = plsc.VectorSubcoreMesh(
        core_axis_name="c", subcore_axis_name="s", num_cores=NC, num_subcores=NS)

    @functools.partial(
        pl.kernel,
        out_type=jax.ShapeDtypeStruct((NC, ACC_ROWS, D), jnp.float32),
        mesh=mesh,
        scratch_types=[
            pltpu.VMEM((SB // 2, CH), jnp.int32),
            pltpu.VMEM((SB // 2, CH), jnp.int32),
            pltpu.VMEM((SB // 2, CH), jnp.int32),
            pltpu.VMEM((CH, D), jnp.float32),
            pltpu.VMEM((CH, D), jnp.float32),
            pltpu.VMEM_SHARED((ACC_ROWS, D), jnp.float32),
            pltpu.VMEM_SHARED((REL_ROWS, D), jnp.float32),
            pltpu.SemaphoreType.DMA,
            pltpu.SemaphoreType.DMA,
        ],
    )
    def k(x_hbm, negrel_hbm, ei_hbm, et_hbm, z_hbm, out_hbm,
          srcv, etv, dstv, rows0, rows1, acc, nrel, sem0, sem1):
        c = lax.axis_index("c")
        s = lax.axis_index("s")
        wid = c * NS + s
        base = wid * T_CHUNKS

        pltpu.sync_copy(z_hbm, rows0)
        for t in range(ROWS_PER_TILE // CH):
            pltpu.sync_copy(rows0,
                            acc.at[pl.ds(s * ROWS_PER_TILE + t * CH, CH)])
        pltpu.sync_copy(
            rows0.at[pl.ds(0, ROWS_PER_TILE % CH)],
            acc.at[pl.ds(s * ROWS_PER_TILE + (ROWS_PER_TILE // CH) * CH,
                         ROWS_PER_TILE % CH)])

        @pl.when(s == 0)
        def _():
            pltpu.sync_copy(negrel_hbm, nrel)

        plsc.subcore_barrier()

        rows = (rows0, rows1)
        sems = (sem0, sem1)
        gtab = (srcv, etv)

        def issue_gather(stage_base, q, b):
            @pl.when(stage_base + 2 * q + b < NREAL_CHUNKS)
            def _():
                tbl = x_hbm if b == 0 else nrel
                pltpu.async_copy(tbl.at[gtab[b].at[q]], rows[b], sems[b])

        @pl.loop(0, T_CHUNKS // SB)
        def _stage(j):
            stage_base = base + j * SB

            @pl.when(stage_base < NREAL_CHUNKS)
            def _():
                pair_base = wid * (T_CHUNKS // 2) + j * (SB // 2)
                pltpu.sync_copy(ei_hbm.at[0, pl.ds(pair_base, SB // 2)], srcv)
                pltpu.sync_copy(et_hbm.at[pl.ds(pair_base, SB // 2)], etv)
                pltpu.sync_copy(ei_hbm.at[1, pl.ds(pair_base, SB // 2)], dstv)
                issue_gather(stage_base, 0, 0)
                issue_gather(stage_base, 0, 1)

                @pl.loop(0, SB // 2)
                def _pair(q):
                    for b in range(2):
                        @pl.when(stage_base + 2 * q + b < NREAL_CHUNKS)
                        def _():
                            pltpu.make_async_copy(
                                x_hbm.at[pl.ds(0, CH)], rows[b],
                                sems[b]).wait()
                            pltpu.sync_copy(rows[b], acc.at[dstv.at[q]],
                                            add=True)

                            @pl.when(q + 1 < SB // 2)
                            def _():
                                issue_gather(stage_base, q + 1, b)

        plsc.subcore_barrier()
        pltpu.sync_copy(
            acc.at[pl.ds(s * ROWS_PER_TILE, ROWS_PER_TILE)],
            out_hbm.at[c, pl.ds(s * ROWS_PER_TILE, ROWS_PER_TILE)])

    return k(x, negrel, ei3, et2, zblk)


def _tc_matmuls(x, acc, weight, bias2d, rel_pad, rel_weight):
    BM = 1000

    def body(x_ref, a0_ref, a1_ref, w_ref, b_ref, r_ref, rw_ref,
             o_ref, ro_ref):
        s = x_ref[...] + a0_ref[0] + a1_ref[0]
        o_ref[...] = jnp.dot(s, w_ref[...],
                             preferred_element_type=jnp.float32) + b_ref[...]
        ro_ref[...] = jnp.dot(r_ref[...], rw_ref[...],
                              preferred_element_type=jnp.float32)

    return pl.pallas_call(
        body,
        grid=(N // BM,),
        in_specs=[
            pl.BlockSpec((BM, D), lambda i: (i, 0)),
            pl.BlockSpec((1, BM, D), lambda i: (0, i, 0)),
            pl.BlockSpec((1, BM, D), lambda i: (1, i, 0)),
            pl.BlockSpec((D, D), lambda i: (0, 0)),
            pl.BlockSpec((1, D), lambda i: (0, 0)),
            pl.BlockSpec((208, D), lambda i: (0, 0)),
            pl.BlockSpec((D, D), lambda i: (0, 0)),
        ],
        out_specs=[
            pl.BlockSpec((BM, D), lambda i: (i, 0)),
            pl.BlockSpec((208, D), lambda i: (0, 0)),
        ],
        out_shape=[
            jax.ShapeDtypeStruct((N, D), jnp.float32),
            jax.ShapeDtypeStruct((208, D), jnp.float32),
        ],
    )(x, acc, acc, weight, bias2d, rel_pad, rel_weight)


def kernel(x, edge_index, edge_type, rel_embed, weight, rel_weight, bias):
    negrel = jnp.concatenate(
        [-rel_embed, jnp.zeros((REL_ROWS - NUM_REL_ROWS, D), jnp.float32)])

    ei3 = edge_index.reshape(2, NEC, CH)
    et2 = edge_type.reshape(NEC, CH)
    zblk = jnp.zeros((CH, D), jnp.float32)
    acc = _sc_scatter(x, negrel, ei3, et2, zblk)

    rel_pad = jnp.concatenate([rel_embed, jnp.zeros((8, D), jnp.float32)])
    out, rel_out = _tc_matmuls(x, acc, weight, bias.reshape(1, D),
                               rel_pad, rel_weight)
    return (out, rel_out[:NUM_REL_ROWS + 1])
